# Initial kernel scaffold; baseline (speedup 1.0000x reference)
import functools

import jax
import jax.numpy as jnp
from jax import lax
from jax.experimental import pallas as pl
from jax.experimental.pallas import tpu as pltpu
from jax.experimental.pallas import tpu_sc as plsc

N = 10000
E = 320000
D = 128
DSUB = 8
NW = 32
NODE_R = 313
CHUNK = 2000
NCHUNK = E // CHUNK
PB = 256
CAP = 2304


def _agg_body(src_hbm, trg_hbm, h_hbm, out_hbm,
              src_c, trg_c, fsrc, ftrg, rows, acc, sem):
    wid = lax.axis_index("s") * 2 + lax.axis_index("c")
    lo = wid * NODE_R
    hi = lo + NODE_R

    zero16 = jnp.zeros((16,), jnp.float32)

    def zinit(i, _):
        for kk in range(DSUB):
            acc[i, kk, :] = zero16
        return 0
    lax.fori_loop(0, NODE_R + 1, zinit, 0)

    def do_batch(base):
        pltpu.async_copy(h_hbm.at[ftrg.at[pl.ds(base, PB)]], rows, sem).wait()

        def grp(g, _):
            def edge(j, _):
                s = fsrc[base + g * 16 + j]
                p = g * 16 + j
                for kk in range(DSUB):
                    a = acc[s, kk, :]
                    r = rows[p, kk, :]
                    acc[s, kk, :] = jnp.maximum(a, r)
                return 0
            lax.fori_loop(0, 16, edge, 0)
            return 0
        lax.fori_loop(0, PB // 16, grp, 0)

    def chunk_body(c, cnt):
        pltpu.sync_copy(src_hbm.at[pl.ds(c * CHUNK, CHUNK)], src_c)
        pltpu.sync_copy(trg_hbm.at[pl.ds(c * CHUNK, CHUNK)], trg_c)

        def filt(i, cnt):
            s = src_c[pl.ds(i * 16, 16)]
            t = trg_c[pl.ds(i * 16, 16)]
            m = (s >= lo) & (s < hi)
            plsc.store_compressed(fsrc.at[pl.ds(cnt, 16)], s - lo, m)
            plsc.store_compressed(ftrg.at[pl.ds(cnt, 16)], t, m)
            pop = plsc.all_reduce_population_count(m)
            return cnt + pop[0]
        cnt = lax.fori_loop(0, CHUNK // 16, filt, cnt)

        nb = cnt // PB

        def proc(b, _):
            do_batch(b * PB)
            return 0
        lax.fori_loop(0, nb, proc, 0)

        def mv(i, _):
            fsrc[pl.ds(i * 16, 16)] = fsrc[pl.ds(nb * PB + i * 16, 16)]
            ftrg[pl.ds(i * 16, 16)] = ftrg[pl.ds(nb * PB + i * 16, 16)]
            return 0
        lax.fori_loop(0, PB // 16, mv, 0)
        return cnt - nb * PB

    cnt = lax.fori_loop(0, NCHUNK, chunk_body, jnp.int32(0))

    junk = jnp.full((16,), NODE_R, jnp.int32)
    zeroi = jnp.zeros((16,), jnp.int32)

    def pad(i, _):
        fsrc[pl.ds(cnt + i * 16, 16)] = junk
        ftrg[pl.ds(cnt + i * 16, 16)] = zeroi
        return 0
    lax.fori_loop(0, PB // 16, pad, 0)

    nb_last = (cnt + PB - 1) // PB

    def proc_last(b, _):
        do_batch(b * PB)
        return 0
    lax.fori_loop(0, nb_last, proc_last, 0)

    @pl.when(wid < NW - 1)
    def _():
        pltpu.sync_copy(acc.at[pl.ds(0, NODE_R)], out_hbm.at[pl.ds(lo, NODE_R)])

    @pl.when(wid == NW - 1)
    def _():
        last = N - (NW - 1) * NODE_R
        pltpu.sync_copy(acc.at[pl.ds(0, last)], out_hbm.at[pl.ds(lo, last)])


_agg_call = functools.partial(
    pl.kernel,
    out_type=jax.ShapeDtypeStruct((N, DSUB, 16), jnp.float32),
    mesh=plsc.VectorSubcoreMesh(core_axis_name="c", subcore_axis_name="s"),
    scratch_types=[
        pltpu.VMEM((CHUNK,), jnp.int32),
        pltpu.VMEM((CHUNK,), jnp.int32),
        pltpu.VMEM((CAP,), jnp.int32),
        pltpu.VMEM((CAP,), jnp.int32),
        pltpu.VMEM((PB, DSUB, 16), jnp.float32),
        pltpu.VMEM((NODE_R + 1, DSUB, 16), jnp.float32),
        pltpu.SemaphoreType.DMA,
    ],
)(_agg_body)


def _h_kernel(x_ref, w_ref, b_ref, h_ref):
    acc = jnp.dot(x_ref[:], w_ref[:], preferred_element_type=jnp.float32,
                  precision=lax.Precision.HIGHEST)
    h_ref[:] = jnp.maximum(acc + b_ref[:], 0.0)


def _out_kernel(x_ref, agg_ref, w1_ref, w2_ref, o_ref):
    o_ref[:] = (
        jnp.dot(x_ref[:], w1_ref[:], preferred_element_type=jnp.float32,
                precision=lax.Precision.HIGHEST)
        + jnp.dot(agg_ref[:], w2_ref[:], preferred_element_type=jnp.float32,
                  precision=lax.Precision.HIGHEST)
    )


_BLK = 1000


def kernel(input_matrix, adjacency_coo_matrix, W_fc, b_fc, W_mat):
    x = input_matrix
    src = adjacency_coo_matrix[0]
    trg = adjacency_coo_matrix[1]

    h = pl.pallas_call(
        _h_kernel,
        grid=(N // _BLK,),
        in_specs=[
            pl.BlockSpec((_BLK, D), lambda i: (i, 0)),
            pl.BlockSpec((D, D), lambda i: (0, 0)),
            pl.BlockSpec((D,), lambda i: (0,)),
        ],
        out_specs=pl.BlockSpec((_BLK, D), lambda i: (i, 0)),
        out_shape=jax.ShapeDtypeStruct((N, D), jnp.float32),
    )(x, W_fc, b_fc)

    agg = _agg_call(src, trg, h.reshape(N, DSUB, 16)).reshape(N, D)

    out = pl.pallas_call(
        _out_kernel,
        grid=(N // _BLK,),
        in_specs=[
            pl.BlockSpec((_BLK, D), lambda i: (i, 0)),
            pl.BlockSpec((_BLK, D), lambda i: (i, 0)),
            pl.BlockSpec((D, D), lambda i: (0, 0)),
            pl.BlockSpec((D, D), lambda i: (0, 0)),
        ],
        out_specs=pl.BlockSpec((_BLK, D), lambda i: (i, 0)),
        out_shape=jax.ShapeDtypeStruct((N, D), jnp.float32),
    )(x, agg, W_mat[:D], W_mat[D:])

    return out


# speedup vs baseline: 1.9340x; 1.9340x over previous
import functools

import jax
import jax.numpy as jnp
from jax import lax
from jax.experimental import pallas as pl
from jax.experimental.pallas import tpu as pltpu
from jax.experimental.pallas import tpu_sc as plsc

N = 10000
E = 320000
D = 128
DSUB = 8
NW = 32
NODE_R = 320
CHUNK = 2000
NCHUNK = E // CHUNK
PB = 256
CAP = 2304


def _agg_body(src_hbm, trg_hbm, h_hbm, out_hbm,
              src_c, trg_c, fsrc, ftrg, rows, acc, sem):
    wid = lax.axis_index("s") * 2 + lax.axis_index("c")
    lo = wid * NODE_R
    hi = lo + NODE_R

    zero16 = jnp.zeros((16,), jnp.float32)

    def zinit(i, _):
        for kk in range(DSUB):
            acc[i, pl.ds(kk * 16, 16)] = zero16
        return 0
    lax.fori_loop(0, NODE_R + 1, zinit, 0)

    def do_batch(base):
        pltpu.async_copy(h_hbm.at[ftrg.at[pl.ds(base, PB)]], rows, sem).wait()

        def grp(g, _):
            sv = fsrc[pl.ds(base + g * 16, 16)]
            for j in range(16):
                s = sv[j]
                p = g * 16 + j
                for kk in range(DSUB):
                    a = acc[s, pl.ds(kk * 16, 16)]
                    r = rows[p, pl.ds(kk * 16, 16)]
                    acc[s, pl.ds(kk * 16, 16)] = jnp.maximum(a, r)
            return 0
        lax.fori_loop(0, PB // 16, grp, 0)

    def chunk_body(c, cnt):
        pltpu.sync_copy(src_hbm.at[pl.ds(c * CHUNK, CHUNK)], src_c)
        pltpu.sync_copy(trg_hbm.at[pl.ds(c * CHUNK, CHUNK)], trg_c)

        def filt(i, cnt):
            s = src_c[pl.ds(i * 16, 16)]
            t = trg_c[pl.ds(i * 16, 16)]
            m = (s >= lo) & (s < hi)
            plsc.store_compressed(fsrc.at[pl.ds(cnt, 16)], s - lo, mask=m)
            plsc.store_compressed(ftrg.at[pl.ds(cnt, 16)], t, mask=m)
            return cnt + jnp.sum(m.astype(jnp.int32))
        cnt = lax.fori_loop(0, CHUNK // 16, filt, cnt)

        nb = cnt // PB

        def proc(b, _):
            do_batch(b * PB)
            return 0
        lax.fori_loop(0, nb, proc, 0)

        def mv(i, _):
            fsrc[pl.ds(i * 16, 16)] = fsrc[pl.ds(nb * PB + i * 16, 16)]
            ftrg[pl.ds(i * 16, 16)] = ftrg[pl.ds(nb * PB + i * 16, 16)]
            return 0
        lax.fori_loop(0, PB // 16, mv, 0)
        return cnt - nb * PB

    cnt = lax.fori_loop(0, NCHUNK, chunk_body, jnp.int32(0))

    junk = jnp.full((16,), NODE_R, jnp.int32)
    zeroi = jnp.zeros((16,), jnp.int32)

    def pad(i, _):
        fsrc[pl.ds(cnt + i * 16, 16)] = junk
        ftrg[pl.ds(cnt + i * 16, 16)] = zeroi
        return 0
    lax.fori_loop(0, PB // 16, pad, 0)

    nb_last = (cnt + PB - 1) // PB

    def proc_last(b, _):
        do_batch(b * PB)
        return 0
    lax.fori_loop(0, nb_last, proc_last, 0)

    @pl.when(wid < NW - 1)
    def _():
        pltpu.sync_copy(acc.at[pl.ds(0, NODE_R)], out_hbm.at[pl.ds(lo, NODE_R)])

    @pl.when(wid == NW - 1)
    def _():
        last = N - (NW - 1) * NODE_R
        pltpu.sync_copy(acc.at[pl.ds(0, last)], out_hbm.at[pl.ds(lo, last)])


_agg_call = functools.partial(
    pl.kernel,
    out_type=jax.ShapeDtypeStruct((N, D), jnp.float32),
    mesh=plsc.VectorSubcoreMesh(core_axis_name="c", subcore_axis_name="s"),
    scratch_types=[
        pltpu.VMEM((CHUNK,), jnp.int32),
        pltpu.VMEM((CHUNK,), jnp.int32),
        pltpu.VMEM((CAP,), jnp.int32),
        pltpu.VMEM((CAP,), jnp.int32),
        pltpu.VMEM((PB, D), jnp.float32),
        pltpu.VMEM((NODE_R + 1, D), jnp.float32),
        pltpu.SemaphoreType.DMA,
    ],
    compiler_params=pltpu.CompilerParams(needs_layout_passes=False),
)(_agg_body)


def _h_kernel(x_ref, w_ref, b_ref, h_ref):
    acc = jnp.dot(x_ref[:], w_ref[:], preferred_element_type=jnp.float32,
                  precision=lax.Precision.HIGHEST)
    h_ref[:] = jnp.maximum(acc + b_ref[:], 0.0)


def _out_kernel(x_ref, agg_ref, w1_ref, w2_ref, o_ref):
    o_ref[:] = (
        jnp.dot(x_ref[:], w1_ref[:], preferred_element_type=jnp.float32,
                precision=lax.Precision.HIGHEST)
        + jnp.dot(agg_ref[:], w2_ref[:], preferred_element_type=jnp.float32,
                  precision=lax.Precision.HIGHEST)
    )


_BLK = 1000


def kernel(input_matrix, adjacency_coo_matrix, W_fc, b_fc, W_mat):
    x = input_matrix
    src = adjacency_coo_matrix[0]
    trg = adjacency_coo_matrix[1]

    h = pl.pallas_call(
        _h_kernel,
        grid=(N // _BLK,),
        in_specs=[
            pl.BlockSpec((_BLK, D), lambda i: (i, 0)),
            pl.BlockSpec((D, D), lambda i: (0, 0)),
            pl.BlockSpec((D,), lambda i: (0,)),
        ],
        out_specs=pl.BlockSpec((_BLK, D), lambda i: (i, 0)),
        out_shape=jax.ShapeDtypeStruct((N, D), jnp.float32),
    )(x, W_fc, b_fc)

    agg = _agg_call(src, trg, h)

    out = pl.pallas_call(
        _out_kernel,
        grid=(N // _BLK,),
        in_specs=[
            pl.BlockSpec((_BLK, D), lambda i: (i, 0)),
            pl.BlockSpec((_BLK, D), lambda i: (i, 0)),
            pl.BlockSpec((D, D), lambda i: (0, 0)),
            pl.BlockSpec((D, D), lambda i: (0, 0)),
        ],
        out_specs=pl.BlockSpec((_BLK, D), lambda i: (i, 0)),
        out_shape=jax.ShapeDtypeStruct((N, D), jnp.float32),
    )(x, agg, W_mat[:D], W_mat[D:])

    return out


# speedup vs baseline: 2.5289x; 1.3076x over previous
import functools

import jax
import jax.numpy as jnp
from jax import lax
from jax.experimental import pallas as pl
from jax.experimental.pallas import tpu as pltpu
from jax.experimental.pallas import tpu_sc as plsc

N = 10000
E = 320000
D = 128
DSUB = 8
NW = 32
NODE_R = 320
CHUNK = 4000
NCHUNK = E // CHUNK
PB = 256
CAP = 4544


def _agg_body(src_hbm, trg_hbm, h_hbm, out_hbm,
              src_c0, src_c1, trg_c0, trg_c1, fsrc, ftrg, rows, acc,
              semc0, semc1, semg):
    wid = lax.axis_index("s") * 2 + lax.axis_index("c")
    lo = wid * NODE_R
    hi = lo + NODE_R
    semc = [semc0, semc1]
    src_c = [src_c0, src_c1]
    trg_c = [trg_c0, trg_c1]

    zero16 = jnp.zeros((16,), jnp.float32)

    def zinit(i, _):
        for kk in range(DSUB):
            acc[i, pl.ds(kk * 16, 16)] = zero16
        return 0
    lax.fori_loop(0, NODE_R + 1, zinit, 0)

    def start_chunk(c, b):
        pltpu.async_copy(src_hbm.at[pl.ds(c * CHUNK, CHUNK)], src_c[b], semc[b])
        pltpu.async_copy(trg_hbm.at[pl.ds(c * CHUNK, CHUNK)], trg_c[b], semc[b])

    def wait_chunk(c, b):
        pltpu.make_async_copy(src_hbm.at[pl.ds(c * CHUNK, CHUNK)], src_c[b],
                              semc[b]).wait()
        pltpu.make_async_copy(trg_hbm.at[pl.ds(c * CHUNK, CHUNK)], trg_c[b],
                              semc[b]).wait()

    def start_gather():
        pltpu.async_copy(h_hbm.at[ftrg.at[pl.ds(0, PB)]], rows, semg)

    def wait_gather():
        pltpu.make_async_copy(h_hbm.at[ftrg.at[pl.ds(0, PB)]], rows, semg).wait()

    def rmw():
        def grp(g, _):
            sv = fsrc[pl.ds(g * 16, 16)]
            for j in range(16):
                s = sv[j]
                p = g * 16 + j
                for kk in range(DSUB):
                    a = acc[s, pl.ds(kk * 16, 16)]
                    r = rows[p, pl.ds(kk * 16, 16)]
                    acc[s, pl.ds(kk * 16, 16)] = jnp.maximum(a, r)
            return 0
        lax.fori_loop(0, PB // 16, grp, 0)

    def move(cnt):
        nmv = (cnt - PB + 15) // 16

        def mv(i, _):
            fsrc[pl.ds(i * 16, 16)] = fsrc[pl.ds(PB + i * 16, 16)]
            ftrg[pl.ds(i * 16, 16)] = ftrg[pl.ds(PB + i * 16, 16)]
            return 0
        lax.fori_loop(0, nmv, mv, 0)

    def finish_batch(cnt):
        wait_gather()
        rmw()
        move(cnt)
        return cnt - PB

    def sync_batch(cnt):
        start_gather()
        return finish_batch(cnt)

    def pair_body(pp, carry):
        cnt, infl = carry
        for b in range(2):
            c = pp * 2 + b
            wait_chunk(c, b)

            def filt(i, cnt):
                s = src_c[b][pl.ds(i * 16, 16)]
                t = trg_c[b][pl.ds(i * 16, 16)]
                m = (s >= lo) & (s < hi)
                plsc.store_compressed(fsrc.at[pl.ds(cnt, 16)], s - lo, mask=m)
                plsc.store_compressed(ftrg.at[pl.ds(cnt, 16)], t, mask=m)
                pop = plsc.all_reduce_population_count(m)
                return cnt + pop[0]
            cnt = lax.fori_loop(0, CHUNK // 16, filt, cnt)

            cnt = lax.cond(infl == 1, finish_batch, lambda cnt: cnt, cnt)
            cnt = lax.while_loop(lambda cnt: cnt >= 2 * PB, sync_batch, cnt)

            @pl.when(cnt >= PB)
            def _():
                start_gather()
            infl = jnp.where(cnt >= PB, jnp.int32(1), jnp.int32(0))

            @pl.when(c + 2 < NCHUNK)
            def _():
                start_chunk(c + 2, b)
        return (cnt, infl)

    start_chunk(0, 0)
    start_chunk(1, 1)
    cnt, infl = lax.fori_loop(0, NCHUNK // 2, pair_body,
                              (jnp.int32(0), jnp.int32(0)))
    cnt = lax.cond(infl == 1, finish_batch, lambda cnt: cnt, cnt)

    junk = jnp.full((16,), NODE_R, jnp.int32)
    zeroi = jnp.zeros((16,), jnp.int32)

    def drain(cnt):
        def pad(i, _):
            fsrc[pl.ds(cnt + i * 16, 16)] = junk
            ftrg[pl.ds(cnt + i * 16, 16)] = zeroi
            return 0
        lax.fori_loop(0, PB // 16, pad, 0)
        return jnp.maximum(sync_batch(cnt), 0)
    cnt = lax.while_loop(lambda cnt: cnt > 0, drain, cnt)

    @pl.when(wid < NW - 1)
    def _():
        pltpu.sync_copy(acc.at[pl.ds(0, NODE_R)], out_hbm.at[pl.ds(lo, NODE_R)])

    @pl.when(wid == NW - 1)
    def _():
        last = N - (NW - 1) * NODE_R
        pltpu.sync_copy(acc.at[pl.ds(0, last)], out_hbm.at[pl.ds(lo, last)])


_agg_call = functools.partial(
    pl.kernel,
    out_type=jax.ShapeDtypeStruct((N, D), jnp.float32),
    mesh=plsc.VectorSubcoreMesh(core_axis_name="c", subcore_axis_name="s"),
    scratch_types=[
        pltpu.VMEM((CHUNK,), jnp.int32),
        pltpu.VMEM((CHUNK,), jnp.int32),
        pltpu.VMEM((CHUNK,), jnp.int32),
        pltpu.VMEM((CHUNK,), jnp.int32),
        pltpu.VMEM((CAP,), jnp.int32),
        pltpu.VMEM((CAP,), jnp.int32),
        pltpu.VMEM((PB, D), jnp.float32),
        pltpu.VMEM((NODE_R + 1, D), jnp.float32),
        pltpu.SemaphoreType.DMA,
        pltpu.SemaphoreType.DMA,
        pltpu.SemaphoreType.DMA,
    ],
    compiler_params=pltpu.CompilerParams(needs_layout_passes=False),
)(_agg_body)


def _h_kernel(x_ref, w_ref, b_ref, h_ref):
    acc = jnp.dot(x_ref[:], w_ref[:], preferred_element_type=jnp.float32,
                  precision=lax.Precision.HIGHEST)
    h_ref[:] = jnp.maximum(acc + b_ref[:], 0.0)


def _out_kernel(x_ref, agg_ref, w1_ref, w2_ref, o_ref):
    o_ref[:] = (
        jnp.dot(x_ref[:], w1_ref[:], preferred_element_type=jnp.float32,
                precision=lax.Precision.HIGHEST)
        + jnp.dot(agg_ref[:], w2_ref[:], preferred_element_type=jnp.float32,
                  precision=lax.Precision.HIGHEST)
    )


_BLK = 1000


def kernel(input_matrix, adjacency_coo_matrix, W_fc, b_fc, W_mat):
    x = input_matrix
    src = adjacency_coo_matrix[0]
    trg = adjacency_coo_matrix[1]

    h = pl.pallas_call(
        _h_kernel,
        grid=(N // _BLK,),
        in_specs=[
            pl.BlockSpec((_BLK, D), lambda i: (i, 0)),
            pl.BlockSpec((D, D), lambda i: (0, 0)),
            pl.BlockSpec((D,), lambda i: (0,)),
        ],
        out_specs=pl.BlockSpec((_BLK, D), lambda i: (i, 0)),
        out_shape=jax.ShapeDtypeStruct((N, D), jnp.float32),
    )(x, W_fc, b_fc)

    agg = _agg_call(src, trg, h)

    out = pl.pallas_call(
        _out_kernel,
        grid=(N // _BLK,),
        in_specs=[
            pl.BlockSpec((_BLK, D), lambda i: (i, 0)),
            pl.BlockSpec((_BLK, D), lambda i: (i, 0)),
            pl.BlockSpec((D, D), lambda i: (0, 0)),
            pl.BlockSpec((D, D), lambda i: (0, 0)),
        ],
        out_specs=pl.BlockSpec((_BLK, D), lambda i: (i, 0)),
        out_shape=jax.ShapeDtypeStruct((N, D), jnp.float32),
    )(x, agg, W_mat[:D], W_mat[D:])

    return out


# speedup vs baseline: 2.6506x; 1.0481x over previous
import functools

import jax
import jax.numpy as jnp
from jax import lax
from jax.experimental import pallas as pl
from jax.experimental.pallas import tpu as pltpu
from jax.experimental.pallas import tpu_sc as plsc

N = 10000
E = 320000
D = 128
DSUB = 8
NW = 32
NODE_R = 320
CHUNK = 4000
NCHUNK = E // CHUNK
PB = 256
CAP = 4544


def _agg_body(src_hbm, trg_hbm, h_hbm, out_hbm,
              src_c0, src_c1, trg_c0, trg_c1, fsrc, ftrg, rows, acc,
              semc0, semc1, semg):
    wid = lax.axis_index("s") * 2 + lax.axis_index("c")
    lo = wid * NODE_R
    hi = lo + NODE_R
    semc = [semc0, semc1]
    src_c = [src_c0, src_c1]
    trg_c = [trg_c0, trg_c1]

    zero16 = jnp.zeros((16,), jnp.float32)

    def zinit(i, _):
        for kk in range(DSUB):
            acc[i, pl.ds(kk * 16, 16)] = zero16
        return 0
    lax.fori_loop(0, NODE_R + 1, zinit, 0)

    def start_chunk(c, b):
        pltpu.async_copy(src_hbm.at[pl.ds(c * CHUNK, CHUNK)], src_c[b], semc[b])
        pltpu.async_copy(trg_hbm.at[pl.ds(c * CHUNK, CHUNK)], trg_c[b], semc[b])

    def wait_chunk(c, b):
        pltpu.make_async_copy(src_hbm.at[pl.ds(c * CHUNK, CHUNK)], src_c[b],
                              semc[b]).wait()
        pltpu.make_async_copy(trg_hbm.at[pl.ds(c * CHUNK, CHUNK)], trg_c[b],
                              semc[b]).wait()

    def start_gather():
        pltpu.async_copy(h_hbm.at[ftrg.at[pl.ds(0, PB)]], rows, semg)

    def wait_gather():
        pltpu.make_async_copy(h_hbm.at[ftrg.at[pl.ds(0, PB)]], rows, semg).wait()

    def rmw():
        def grp(g, _):
            sv = fsrc[pl.ds(g * 16, 16)]
            for j in range(16):
                s = sv[j]
                p = g * 16 + j
                a = [acc[s, pl.ds(kk * 16, 16)] for kk in range(DSUB)]
                r = [rows[p, pl.ds(kk * 16, 16)] for kk in range(DSUB)]
                mx = [jnp.maximum(a[kk], r[kk]) for kk in range(DSUB)]
                for kk in range(DSUB):
                    acc[s, pl.ds(kk * 16, 16)] = mx[kk]
            return 0
        lax.fori_loop(0, PB // 16, grp, 0)

    def move(cnt):
        nmv = (cnt - PB + 15) // 16

        def mv(i, _):
            fsrc[pl.ds(i * 16, 16)] = fsrc[pl.ds(PB + i * 16, 16)]
            ftrg[pl.ds(i * 16, 16)] = ftrg[pl.ds(PB + i * 16, 16)]
            return 0
        lax.fori_loop(0, nmv, mv, 0)

    def finish_batch(cnt):
        wait_gather()
        rmw()
        move(cnt)
        return cnt - PB

    def sync_batch(cnt):
        start_gather()
        return finish_batch(cnt)

    def pair_body(pp, carry):
        cnt, infl = carry
        for b in range(2):
            c = pp * 2 + b
            wait_chunk(c, b)

            def filt(i, cnt):
                s = src_c[b][pl.ds(i * 16, 16)]
                t = trg_c[b][pl.ds(i * 16, 16)]
                m = (s >= lo) & (s < hi)
                pop = plsc.all_reduce_population_count(m)
                npos = pop[0]

                @pl.when(npos > 0)
                def _():
                    plsc.store_compressed(fsrc.at[pl.ds(cnt, 16)], s - lo, mask=m)
                    plsc.store_compressed(ftrg.at[pl.ds(cnt, 16)], t, mask=m)
                return cnt + npos
            cnt = lax.fori_loop(0, CHUNK // 16, filt, cnt)

            cnt = lax.cond(infl == 1, finish_batch, lambda cnt: cnt, cnt)
            cnt = lax.while_loop(lambda cnt: cnt >= 2 * PB, sync_batch, cnt)

            @pl.when(cnt >= PB)
            def _():
                start_gather()
            infl = jnp.where(cnt >= PB, jnp.int32(1), jnp.int32(0))

            @pl.when(c + 2 < NCHUNK)
            def _():
                start_chunk(c + 2, b)
        return (cnt, infl)

    start_chunk(0, 0)
    start_chunk(1, 1)
    cnt, infl = lax.fori_loop(0, NCHUNK // 2, pair_body,
                              (jnp.int32(0), jnp.int32(0)))
    cnt = lax.cond(infl == 1, finish_batch, lambda cnt: cnt, cnt)

    junk = jnp.full((16,), NODE_R, jnp.int32)
    zeroi = jnp.zeros((16,), jnp.int32)

    def drain(cnt):
        def pad(i, _):
            fsrc[pl.ds(cnt + i * 16, 16)] = junk
            ftrg[pl.ds(cnt + i * 16, 16)] = zeroi
            return 0
        lax.fori_loop(0, PB // 16, pad, 0)
        return jnp.maximum(sync_batch(cnt), 0)
    cnt = lax.while_loop(lambda cnt: cnt > 0, drain, cnt)

    @pl.when(wid < NW - 1)
    def _():
        pltpu.sync_copy(acc.at[pl.ds(0, NODE_R)], out_hbm.at[pl.ds(lo, NODE_R)])

    @pl.when(wid == NW - 1)
    def _():
        last = N - (NW - 1) * NODE_R
        pltpu.sync_copy(acc.at[pl.ds(0, last)], out_hbm.at[pl.ds(lo, last)])


_agg_call = functools.partial(
    pl.kernel,
    out_type=jax.ShapeDtypeStruct((N, D), jnp.float32),
    mesh=plsc.VectorSubcoreMesh(core_axis_name="c", subcore_axis_name="s"),
    scratch_types=[
        pltpu.VMEM((CHUNK,), jnp.int32),
        pltpu.VMEM((CHUNK,), jnp.int32),
        pltpu.VMEM((CHUNK,), jnp.int32),
        pltpu.VMEM((CHUNK,), jnp.int32),
        pltpu.VMEM((CAP,), jnp.int32),
        pltpu.VMEM((CAP,), jnp.int32),
        pltpu.VMEM((PB, D), jnp.float32),
        pltpu.VMEM((NODE_R + 1, D), jnp.float32),
        pltpu.SemaphoreType.DMA,
        pltpu.SemaphoreType.DMA,
        pltpu.SemaphoreType.DMA,
    ],
    compiler_params=pltpu.CompilerParams(needs_layout_passes=False),
)(_agg_body)


def _h_kernel(x_ref, w_ref, b_ref, h_ref):
    acc = jnp.dot(x_ref[:], w_ref[:], preferred_element_type=jnp.float32,
                  precision=lax.Precision.HIGHEST)
    h_ref[:] = jnp.maximum(acc + b_ref[:], 0.0)


def _out_kernel(x_ref, agg_ref, w1_ref, w2_ref, o_ref):
    o_ref[:] = (
        jnp.dot(x_ref[:], w1_ref[:], preferred_element_type=jnp.float32,
                precision=lax.Precision.HIGHEST)
        + jnp.dot(agg_ref[:], w2_ref[:], preferred_element_type=jnp.float32,
                  precision=lax.Precision.HIGHEST)
    )


_BLK = 1000


def kernel(input_matrix, adjacency_coo_matrix, W_fc, b_fc, W_mat):
    x = input_matrix
    src = adjacency_coo_matrix[0]
    trg = adjacency_coo_matrix[1]

    h = pl.pallas_call(
        _h_kernel,
        grid=(N // _BLK,),
        in_specs=[
            pl.BlockSpec((_BLK, D), lambda i: (i, 0)),
            pl.BlockSpec((D, D), lambda i: (0, 0)),
            pl.BlockSpec((D,), lambda i: (0,)),
        ],
        out_specs=pl.BlockSpec((_BLK, D), lambda i: (i, 0)),
        out_shape=jax.ShapeDtypeStruct((N, D), jnp.float32),
    )(x, W_fc, b_fc)

    agg = _agg_call(src, trg, h)

    out = pl.pallas_call(
        _out_kernel,
        grid=(N // _BLK,),
        in_specs=[
            pl.BlockSpec((_BLK, D), lambda i: (i, 0)),
            pl.BlockSpec((_BLK, D), lambda i: (i, 0)),
            pl.BlockSpec((D, D), lambda i: (0, 0)),
            pl.BlockSpec((D, D), lambda i: (0, 0)),
        ],
        out_specs=pl.BlockSpec((_BLK, D), lambda i: (i, 0)),
        out_shape=jax.ShapeDtypeStruct((N, D), jnp.float32),
    )(x, agg, W_mat[:D], W_mat[D:])

    return out


# speedup vs baseline: 3.2290x; 1.2182x over previous
import functools

import jax
import jax.numpy as jnp
from jax import lax
from jax.experimental import pallas as pl
from jax.experimental.pallas import tpu as pltpu
from jax.experimental.pallas import tpu_sc as plsc

N = 10000
E = 320000
D = 128
DSUB = 8
NW = 32
NODE_R = 320
CHUNK = 4000
NCHUNK = E // CHUNK
PB = 256
CAP = 4544


def _agg_body(src_hbm, trg_hbm, h_hbm, out_hbm,
              src_c0, src_c1, trg_c0, trg_c1, fsrc, ftrg, rows, acc,
              semc0, semc1, semg):
    wid = lax.axis_index("s") * 2 + lax.axis_index("c")
    lo = wid * NODE_R
    hi = lo + NODE_R
    semc = [semc0, semc1]
    src_c = [src_c0, src_c1]
    trg_c = [trg_c0, trg_c1]

    zero16 = jnp.zeros((16,), jnp.float32)

    def zinit(i, _):
        for kk in range(DSUB):
            acc[i, pl.ds(kk * 16, 16)] = zero16
        return 0
    lax.fori_loop(0, NODE_R + 1, zinit, 0)

    def start_chunk(c, b):
        pltpu.async_copy(src_hbm.at[pl.ds(c * CHUNK, CHUNK)], src_c[b], semc[b])
        pltpu.async_copy(trg_hbm.at[pl.ds(c * CHUNK, CHUNK)], trg_c[b], semc[b])

    def wait_chunk(c, b):
        pltpu.make_async_copy(src_hbm.at[pl.ds(c * CHUNK, CHUNK)], src_c[b],
                              semc[b]).wait()
        pltpu.make_async_copy(trg_hbm.at[pl.ds(c * CHUNK, CHUNK)], trg_c[b],
                              semc[b]).wait()

    def start_gather():
        pltpu.async_copy(h_hbm.at[ftrg.at[pl.ds(0, PB)]], rows, semg)

    def wait_gather():
        pltpu.make_async_copy(h_hbm.at[ftrg.at[pl.ds(0, PB)]], rows, semg).wait()

    def rmw():
        def grp(g, _):
            sv = fsrc[pl.ds(g * 16, 16)]
            for j in range(16):
                s = sv[j]
                p = g * 16 + j
                a = [acc[s, pl.ds(kk * 16, 16)] for kk in range(DSUB)]
                r = [rows[p, pl.ds(kk * 16, 16)] for kk in range(DSUB)]
                mx = [jnp.maximum(a[kk], r[kk]) for kk in range(DSUB)]
                for kk in range(DSUB):
                    acc[s, pl.ds(kk * 16, 16)] = mx[kk]
            return 0
        lax.fori_loop(0, PB // 16, grp, 0)

    def move(cnt):
        nmv = (cnt - PB + 15) // 16

        def mv(i, _):
            fsrc[pl.ds(i * 16, 16)] = fsrc[pl.ds(PB + i * 16, 16)]
            ftrg[pl.ds(i * 16, 16)] = ftrg[pl.ds(PB + i * 16, 16)]
            return 0
        lax.fori_loop(0, nmv, mv, 0)

    def finish_batch(cnt):
        wait_gather()
        rmw()
        move(cnt)
        return cnt - PB

    def sync_batch(cnt):
        start_gather()
        return finish_batch(cnt)

    def pair_body(pp, carry):
        cnt, infl = carry
        for b in range(2):
            c = pp * 2 + b
            wait_chunk(c, b)

            def filt(i, cntv):
                s = src_c[b][pl.ds(i * 16, 16)]
                t = trg_c[b][pl.ds(i * 16, 16)]
                m = (s >= lo) & (s < hi)
                rank = plsc.cumsum(m.astype(jnp.int32)) - 1
                pos = cntv + rank
                plsc.store_scatter(fsrc, [pos], s - lo, mask=m)
                plsc.store_scatter(ftrg, [pos], t, mask=m)
                return cntv + plsc.all_reduce_population_count(m)
            cntv = lax.fori_loop(0, CHUNK // 16, filt,
                                 jnp.broadcast_to(cnt, (16,)).astype(jnp.int32))
            cnt = cntv[0]

            cnt = lax.cond(infl == 1, finish_batch, lambda cnt: cnt, cnt)
            cnt = lax.while_loop(lambda cnt: cnt >= 2 * PB, sync_batch, cnt)

            @pl.when(cnt >= PB)
            def _():
                start_gather()
            infl = jnp.where(cnt >= PB, jnp.int32(1), jnp.int32(0))

            @pl.when(c + 2 < NCHUNK)
            def _():
                start_chunk(c + 2, b)
        return (cnt, infl)

    start_chunk(0, 0)
    start_chunk(1, 1)
    cnt, infl = lax.fori_loop(0, NCHUNK // 2, pair_body,
                              (jnp.int32(0), jnp.int32(0)))
    cnt = lax.cond(infl == 1, finish_batch, lambda cnt: cnt, cnt)

    junk = jnp.full((16,), NODE_R, jnp.int32)
    zeroi = jnp.zeros((16,), jnp.int32)

    def drain(cnt):
        def pad(i, _):
            fsrc[pl.ds(cnt + i * 16, 16)] = junk
            ftrg[pl.ds(cnt + i * 16, 16)] = zeroi
            return 0
        lax.fori_loop(0, PB // 16, pad, 0)
        return jnp.maximum(sync_batch(cnt), 0)
    cnt = lax.while_loop(lambda cnt: cnt > 0, drain, cnt)

    @pl.when(wid < NW - 1)
    def _():
        pltpu.sync_copy(acc.at[pl.ds(0, NODE_R)], out_hbm.at[pl.ds(lo, NODE_R)])

    @pl.when(wid == NW - 1)
    def _():
        last = N - (NW - 1) * NODE_R
        pltpu.sync_copy(acc.at[pl.ds(0, last)], out_hbm.at[pl.ds(lo, last)])


_agg_call = functools.partial(
    pl.kernel,
    out_type=jax.ShapeDtypeStruct((N, D), jnp.float32),
    mesh=plsc.VectorSubcoreMesh(core_axis_name="c", subcore_axis_name="s"),
    scratch_types=[
        pltpu.VMEM((CHUNK,), jnp.int32),
        pltpu.VMEM((CHUNK,), jnp.int32),
        pltpu.VMEM((CHUNK,), jnp.int32),
        pltpu.VMEM((CHUNK,), jnp.int32),
        pltpu.VMEM((CAP,), jnp.int32),
        pltpu.VMEM((CAP,), jnp.int32),
        pltpu.VMEM((PB, D), jnp.float32),
        pltpu.VMEM((NODE_R + 1, D), jnp.float32),
        pltpu.SemaphoreType.DMA,
        pltpu.SemaphoreType.DMA,
        pltpu.SemaphoreType.DMA,
    ],
    compiler_params=pltpu.CompilerParams(needs_layout_passes=False),
)(_agg_body)


def _h_kernel(x_ref, w_ref, b_ref, h_ref):
    acc = jnp.dot(x_ref[:], w_ref[:], preferred_element_type=jnp.float32,
                  precision=lax.Precision.HIGHEST)
    h_ref[:] = jnp.maximum(acc + b_ref[:], 0.0)


def _out_kernel(x_ref, agg_ref, w1_ref, w2_ref, o_ref):
    o_ref[:] = (
        jnp.dot(x_ref[:], w1_ref[:], preferred_element_type=jnp.float32,
                precision=lax.Precision.HIGHEST)
        + jnp.dot(agg_ref[:], w2_ref[:], preferred_element_type=jnp.float32,
                  precision=lax.Precision.HIGHEST)
    )


_BLK = 1000


def kernel(input_matrix, adjacency_coo_matrix, W_fc, b_fc, W_mat):
    x = input_matrix
    src = adjacency_coo_matrix[0]
    trg = adjacency_coo_matrix[1]

    h = pl.pallas_call(
        _h_kernel,
        grid=(N // _BLK,),
        in_specs=[
            pl.BlockSpec((_BLK, D), lambda i: (i, 0)),
            pl.BlockSpec((D, D), lambda i: (0, 0)),
            pl.BlockSpec((D,), lambda i: (0,)),
        ],
        out_specs=pl.BlockSpec((_BLK, D), lambda i: (i, 0)),
        out_shape=jax.ShapeDtypeStruct((N, D), jnp.float32),
    )(x, W_fc, b_fc)

    agg = _agg_call(src, trg, h)

    out = pl.pallas_call(
        _out_kernel,
        grid=(N // _BLK,),
        in_specs=[
            pl.BlockSpec((_BLK, D), lambda i: (i, 0)),
            pl.BlockSpec((_BLK, D), lambda i: (i, 0)),
            pl.BlockSpec((D, D), lambda i: (0, 0)),
            pl.BlockSpec((D, D), lambda i: (0, 0)),
        ],
        out_specs=pl.BlockSpec((_BLK, D), lambda i: (i, 0)),
        out_shape=jax.ShapeDtypeStruct((N, D), jnp.float32),
    )(x, agg, W_mat[:D], W_mat[D:])

    return out
